# baseline (speedup 1.0000x reference)
import functools

import jax
import jax.numpy as jnp
from jax import lax
from jax.experimental import pallas as pl
from jax.experimental.pallas import tpu as pltpu
from jax.experimental.pallas import tpu_sc as plsc

K = 16
R = 8
CW = 512
NC = 2
NS = 16
LL = 16
QB = 8
MB = 256
IBIG = 2**30


def _knn_body(nch, pos_r, posT3, batT3, batR, out, D):
    qx = pos_r[:, 0:1]
    qy = pos_r[:, 1:2]
    qz = pos_r[:, 2:3]
    qb = batR[:, 0:1]
    i = pl.program_id(0)
    selfi = i * R + lax.broadcasted_iota(jnp.int32, (R, 1), 0)

    def fill(c, _):
        px = posT3[c, 0:1, :]
        py = posT3[c, 1:2, :]
        pz = posT3[c, 2:3, :]
        dx = qx - px
        dy = qy - py
        dz = qz - pz
        d2 = (dx * dx + dy * dy) + dz * dz
        mb = batT3[c] != qb
        D[c] = jnp.where(mb, jnp.inf, d2)
        return 0

    lax.fori_loop(0, nch, fill, 0)

    pv = jnp.full((R, 1), -jnp.inf, jnp.float32)
    pi = jnp.full((R, 1), -1, jnp.int32)
    for t in range(K):
        def scan(c, bc, pv=pv, pi=pi):
            bv, bi = bc
            Dc = D[c]
            ii = c * CW + lax.broadcasted_iota(jnp.int32, (R, CW), 1)
            valid = (Dc > pv) | ((Dc == pv) & (ii > pi))
            v = jnp.where(valid, Dc, jnp.inf)
            m = jnp.min(v, axis=1, keepdims=True)
            im = jnp.min(jnp.where(v == m, ii, IBIG), axis=1, keepdims=True)
            take = m < bv
            return jnp.where(take, m, bv), jnp.where(take, im, bi)

        bv, bi = lax.fori_loop(
            0, nch, scan,
            (jnp.full((R, 1), jnp.inf, jnp.float32),
             jnp.full((R, 1), IBIG, jnp.int32)))
        bi = jnp.where(bv == jnp.inf, selfi, bi)
        out[:, t:t + 1] = bi
        pv, pi = bv, bi


def _knn_topk(pos, batch):
    N = pos.shape[0]
    nch = -(-N // CW)
    npad = nch * CW
    nrp = -(-N // R) * R
    b32 = batch.astype(jnp.int32)
    pos_r = jnp.pad(pos, ((0, nrp - N), (0, 0)))
    batR = jnp.pad(b32, (0, nrp - N), constant_values=126)[:, None]
    posT3 = jnp.pad(pos, ((0, npad - N), (0, 0))).T.reshape(3, nch, CW)
    posT3 = posT3.transpose(1, 0, 2)
    batT3 = jnp.pad(b32, (0, npad - N), constant_values=127)
    batT3 = jnp.broadcast_to(batT3.reshape(nch, 1, CW), (nch, R, CW))
    grid = nrp // R
    return pl.pallas_call(
        functools.partial(_knn_body, nch),
        grid=(grid,),
        in_specs=[
            pl.BlockSpec((R, 3), lambda i: (i, 0)),
            pl.BlockSpec((nch, 3, CW), lambda i: (0, 0, 0)),
            pl.BlockSpec((nch, R, CW), lambda i: (0, 0, 0)),
            pl.BlockSpec((R, 1), lambda i: (i, 0)),
        ],
        out_specs=pl.BlockSpec((R, K), lambda i: (i, 0)),
        out_shape=jax.ShapeDtypeStruct((nrp, K), jnp.int32),
        scratch_shapes=[pltpu.VMEM((nch, R, CW), jnp.float32)],
    )(pos_r, posT3, batT3, batR)[:N]


def _sc_body(nqp, C, x_hbm, idxf_hbm, out_hbm, nbuf, qbuf, idxf, zbuf,
             var_sh, sem):
    c = lax.axis_index("c")
    s = lax.axis_index("s")
    wid = s * NC + c
    rows_per_tile = nqp // NS
    for r in range(LL):
        for ch in range(C // LL):
            zbuf[r, pl.ds(ch * LL, LL)] = jnp.zeros((LL,), jnp.float32)

    def zloop(z, _):
        pltpu.sync_copy(zbuf, var_sh.at[pl.ds(s * rows_per_tile + z * LL, LL)])
        return 0

    lax.fori_loop(0, rows_per_tile // LL, zloop, 0)
    plsc.subcore_barrier()

    qpw = nqp // (NC * NS)

    def giter(g, _):
        base = wid * qpw + g * QB
        pltpu.sync_copy(idxf_hbm.at[pl.ds(base * K, QB * K)], idxf)
        cp = pltpu.async_copy(x_hbm.at[idxf], nbuf, sem)
        pltpu.sync_copy(x_hbm.at[pl.ds(base, QB)], qbuf)
        cp.wait()

        def qloop(q, _):
            for ch in range(C // LL):
                qv = qbuf[q, pl.ds(ch * LL, LL)]
                for j in range(K):
                    d = nbuf[q * K + j, pl.ds(ch * LL, LL)] - qv
                    nbuf[q * K + j, pl.ds(ch * LL, LL)] = d * d
            return 0

        lax.fori_loop(0, QB, qloop, 0)
        pltpu.sync_copy(nbuf, var_sh.at[idxf], add=True)
        return 0

    lax.fori_loop(0, qpw // QB, giter, 0)
    plsc.subcore_barrier()
    pltpu.sync_copy(var_sh.at[pl.ds(s * rows_per_tile, rows_per_tile)],
                    out_hbm.at[c, pl.ds(s * rows_per_tile, rows_per_tile)])


def _sc_variance(x, idx):
    N, C = x.shape
    nqp = -(-N // (NC * NS * QB)) * (NC * NS * QB)
    x_pad = jnp.pad(x, ((0, nqp - N), (0, 0)))
    idx_pad = jnp.pad(idx, ((0, nqp - N), (0, 0)), constant_values=N)
    idx_flat = idx_pad.reshape(-1)
    mesh = plsc.VectorSubcoreMesh(core_axis_name="c", subcore_axis_name="s",
                                  num_cores=NC, num_subcores=NS)
    fn = pl.kernel(
        functools.partial(_sc_body, nqp, C),
        out_type=jax.ShapeDtypeStruct((NC, nqp, C), jnp.float32),
        mesh=mesh,
        scratch_types=[
            pltpu.VMEM((QB * K, C), jnp.float32),
            pltpu.VMEM((QB, C), jnp.float32),
            pltpu.VMEM((QB * K,), jnp.int32),
            pltpu.VMEM((LL, C), jnp.float32),
            pltpu.VMEM_SHARED((nqp, C), jnp.float32),
            pltpu.SemaphoreType.DMA,
        ],
    )
    return fn(x_pad, idx_flat)


def _mlp_body(var2, W1T, b1, w2, b2, out):
    v = (var2[0] + var2[1]) * jnp.float32(1.0 / K)
    h = jnp.dot(v, W1T[...], preferred_element_type=jnp.float32) + b1[...]
    h = jnp.maximum(h, 0.0)
    o = jnp.dot(h, w2[...], preferred_element_type=jnp.float32) + b2[0, 0]
    out[...] = jax.nn.sigmoid(o)


def _mlp(var2, W1, b1, W2, b2):
    _, nqp, C = var2.shape
    H = W1.shape[0]
    grid = nqp // MB
    return pl.pallas_call(
        _mlp_body,
        grid=(grid,),
        in_specs=[
            pl.BlockSpec((NC, MB, C), lambda i: (0, i, 0)),
            pl.BlockSpec((C, H), lambda i: (0, 0)),
            pl.BlockSpec((1, H), lambda i: (0, 0)),
            pl.BlockSpec((H, 1), lambda i: (0, 0)),
            pl.BlockSpec((1, 1), lambda i: (0, 0), memory_space=pltpu.SMEM),
        ],
        out_specs=pl.BlockSpec((MB, 1), lambda i: (i, 0)),
        out_shape=jax.ShapeDtypeStruct((nqp, 1), jnp.float32),
    )(var2, W1.T, b1[None, :], W2.T, b2[None, :])


def kernel(x, pos, batch, W1, b1, W2, b2):
    N = x.shape[0]
    idx = _knn_topk(pos, batch)
    var2 = _sc_variance(x, idx)
    return _mlp(var2, W1, b1, W2, b2)[:N]


# speedup vs baseline: 5.7627x; 5.7627x over previous
import functools

import jax
import jax.numpy as jnp
from jax import lax
from jax.experimental import pallas as pl
from jax.experimental.pallas import tpu as pltpu
from jax.experimental.pallas import tpu_sc as plsc

K = 16
R = 8
CW = 512
NC = 2
NS = 16
LL = 16
QB = 8
MB = 256
IBIG = 2**30


def _knn_body(nch, cl_ref, cn_ref, pos_r, posT3, batT3, batR, out, D):
    qx = pos_r[:, 0:1]
    qy = pos_r[:, 1:2]
    qz = pos_r[:, 2:3]
    qb = batR[:, 0:1]
    i = pl.program_id(0)
    selfi = i * R + lax.broadcasted_iota(jnp.int32, (R, 1), 0)
    clo = cl_ref[i]
    chi = clo + cn_ref[i]

    def fill(c, _):
        px = posT3[c, 0:1, :]
        py = posT3[c, 1:2, :]
        pz = posT3[c, 2:3, :]
        dx = qx - px
        dy = qy - py
        dz = qz - pz
        d2 = (dx * dx + dy * dy) + dz * dz
        mb = batT3[c] != qb
        D[c] = jnp.where(mb, jnp.inf, d2)
        return 0

    lax.fori_loop(clo, chi, fill, 0)

    pv = jnp.full((R, 1), -jnp.inf, jnp.float32)
    pi = jnp.full((R, 1), -1, jnp.int32)
    for t in range(K):
        def scan(c, bc, pv=pv, pi=pi):
            bv, bi = bc
            Dc = D[c]
            ii = c * CW + lax.broadcasted_iota(jnp.int32, (R, CW), 1)
            valid = (Dc > pv) | ((Dc == pv) & (ii > pi))
            v = jnp.where(valid, Dc, jnp.inf)
            m = jnp.min(v, axis=1, keepdims=True)
            im = jnp.min(jnp.where(v == m, ii, IBIG), axis=1, keepdims=True)
            take = m < bv
            return jnp.where(take, m, bv), jnp.where(take, im, bi)

        bv, bi = lax.fori_loop(
            clo, chi, scan,
            (jnp.full((R, 1), jnp.inf, jnp.float32),
             jnp.full((R, 1), IBIG, jnp.int32)))
        bi = jnp.where(bv == jnp.inf, selfi, bi)
        out[:, t:t + 1] = bi
        pv, pi = bv, bi


def _knn_topk(pos, batch):
    N = pos.shape[0]
    nch = -(-N // CW)
    npad = nch * CW
    nrp = -(-N // R) * R
    b32 = batch.astype(jnp.int32)
    pos_r = jnp.pad(pos, ((0, nrp - N), (0, 0)))
    batR = jnp.pad(b32, (0, nrp - N), constant_values=126)[:, None]
    posT3 = jnp.pad(pos, ((0, npad - N), (0, 0))).T.reshape(3, nch, CW)
    posT3 = posT3.transpose(1, 0, 2)
    batT3 = jnp.pad(b32, (0, npad - N), constant_values=127)
    batT3 = jnp.broadcast_to(batT3.reshape(nch, 1, CW), (nch, R, CW))
    grid = nrp // R
    seg = jnp.searchsorted(b32, jnp.arange(9), side="left").astype(jnp.int32)
    rb = jnp.arange(grid, dtype=jnp.int32)
    bpad = jnp.pad(b32, (0, nrp - N), constant_values=7)
    blo = jnp.clip(bpad[rb * R], 0, 7)
    bhi = jnp.clip(bpad[rb * R + R - 1], 0, 7)
    c_lo = seg[blo] // CW
    c_hi = -(-seg[bhi + 1] // CW)
    c_cnt = c_hi - c_lo
    return pl.pallas_call(
        functools.partial(_knn_body, nch),
        grid=(grid,),
        in_specs=[
            pl.BlockSpec(memory_space=pltpu.SMEM),
            pl.BlockSpec(memory_space=pltpu.SMEM),
            pl.BlockSpec((R, 3), lambda i: (i, 0)),
            pl.BlockSpec((nch, 3, CW), lambda i: (0, 0, 0)),
            pl.BlockSpec((nch, R, CW), lambda i: (0, 0, 0)),
            pl.BlockSpec((R, 1), lambda i: (i, 0)),
        ],
        out_specs=pl.BlockSpec((R, K), lambda i: (i, 0)),
        out_shape=jax.ShapeDtypeStruct((nrp, K), jnp.int32),
        scratch_shapes=[pltpu.VMEM((nch, R, CW), jnp.float32)],
    )(c_lo, c_cnt, pos_r, posT3, batT3, batR)[:N]


def _sc_body(nqp, C, x_hbm, idxf_hbm, out_hbm, nbuf, qbuf, idxf, zbuf,
             var_sh, sem):
    c = lax.axis_index("c")
    s = lax.axis_index("s")
    wid = s * NC + c
    rows_per_tile = nqp // NS
    for r in range(LL):
        for ch in range(C // LL):
            zbuf[r, pl.ds(ch * LL, LL)] = jnp.zeros((LL,), jnp.float32)

    def zloop(z, _):
        pltpu.sync_copy(zbuf, var_sh.at[pl.ds(s * rows_per_tile + z * LL, LL)])
        return 0

    lax.fori_loop(0, rows_per_tile // LL, zloop, 0)
    plsc.subcore_barrier()

    qpw = nqp // (NC * NS)

    def giter(g, _):
        base = wid * qpw + g * QB
        pltpu.sync_copy(idxf_hbm.at[pl.ds(base * K, QB * K)], idxf)
        cp = pltpu.async_copy(x_hbm.at[idxf], nbuf, sem)
        pltpu.sync_copy(x_hbm.at[pl.ds(base, QB)], qbuf)
        cp.wait()

        def qloop(q, _):
            for ch in range(C // LL):
                qv = qbuf[q, pl.ds(ch * LL, LL)]
                for j in range(K):
                    d = nbuf[q * K + j, pl.ds(ch * LL, LL)] - qv
                    nbuf[q * K + j, pl.ds(ch * LL, LL)] = d * d
            return 0

        lax.fori_loop(0, QB, qloop, 0)
        pltpu.sync_copy(nbuf, var_sh.at[idxf], add=True)
        return 0

    lax.fori_loop(0, qpw // QB, giter, 0)
    plsc.subcore_barrier()
    pltpu.sync_copy(var_sh.at[pl.ds(s * rows_per_tile, rows_per_tile)],
                    out_hbm.at[c, pl.ds(s * rows_per_tile, rows_per_tile)])


def _sc_variance(x, idx):
    N, C = x.shape
    nqp = -(-N // (NC * NS * QB)) * (NC * NS * QB)
    x_pad = jnp.pad(x, ((0, nqp - N), (0, 0)))
    idx_pad = jnp.pad(idx, ((0, nqp - N), (0, 0)), constant_values=N)
    idx_flat = idx_pad.reshape(-1)
    mesh = plsc.VectorSubcoreMesh(core_axis_name="c", subcore_axis_name="s",
                                  num_cores=NC, num_subcores=NS)
    fn = pl.kernel(
        functools.partial(_sc_body, nqp, C),
        out_type=jax.ShapeDtypeStruct((NC, nqp, C), jnp.float32),
        mesh=mesh,
        scratch_types=[
            pltpu.VMEM((QB * K, C), jnp.float32),
            pltpu.VMEM((QB, C), jnp.float32),
            pltpu.VMEM((QB * K,), jnp.int32),
            pltpu.VMEM((LL, C), jnp.float32),
            pltpu.VMEM_SHARED((nqp, C), jnp.float32),
            pltpu.SemaphoreType.DMA,
        ],
    )
    return fn(x_pad, idx_flat)


def _mlp_body(var2, W1T, b1, w2, b2, out):
    v = (var2[0] + var2[1]) * jnp.float32(1.0 / K)
    h = jnp.dot(v, W1T[...], preferred_element_type=jnp.float32) + b1[...]
    h = jnp.maximum(h, 0.0)
    o = jnp.dot(h, w2[...], preferred_element_type=jnp.float32) + b2[0, 0]
    out[...] = jax.nn.sigmoid(o)


def _mlp(var2, W1, b1, W2, b2):
    _, nqp, C = var2.shape
    H = W1.shape[0]
    grid = nqp // MB
    return pl.pallas_call(
        _mlp_body,
        grid=(grid,),
        in_specs=[
            pl.BlockSpec((NC, MB, C), lambda i: (0, i, 0)),
            pl.BlockSpec((C, H), lambda i: (0, 0)),
            pl.BlockSpec((1, H), lambda i: (0, 0)),
            pl.BlockSpec((H, 1), lambda i: (0, 0)),
            pl.BlockSpec((1, 1), lambda i: (0, 0), memory_space=pltpu.SMEM),
        ],
        out_specs=pl.BlockSpec((MB, 1), lambda i: (i, 0)),
        out_shape=jax.ShapeDtypeStruct((nqp, 1), jnp.float32),
    )(var2, W1.T, b1[None, :], W2.T, b2[None, :])


def kernel(x, pos, batch, W1, b1, W2, b2):
    N = x.shape[0]
    idx = _knn_topk(pos, batch)
    var2 = _sc_variance(x, idx)
    return _mlp(var2, W1, b1, W2, b2)[:N]


# speedup vs baseline: 19.8597x; 3.4462x over previous
import functools

import jax
import jax.numpy as jnp
from jax import lax
from jax.experimental import pallas as pl
from jax.experimental.pallas import tpu as pltpu
from jax.experimental.pallas import tpu_sc as plsc

K = 16
R = 32
CW = 512
NC = 2
NS = 16
LL = 16
QB = 8
MB = 256
IBIG = 2**30


def _knn_body(nch, cl_ref, cn_ref, pos_r, posT3, batT3, batR, out, D):
    qx = pos_r[:, 0:1]
    qy = pos_r[:, 1:2]
    qz = pos_r[:, 2:3]
    qb = batR[:, 0:1]
    i = pl.program_id(0)
    selfi = i * R + lax.broadcasted_iota(jnp.int32, (R, 1), 0)
    clo = cl_ref[i]
    chi = clo + cn_ref[i]

    def fill(c, _):
        px = posT3[c, 0:1, :]
        py = posT3[c, 1:2, :]
        pz = posT3[c, 2:3, :]
        dx = qx - px
        dy = qy - py
        dz = qz - pz
        d2 = (dx * dx + dy * dy) + dz * dz
        mb = batT3[c] != qb
        D[c] = jnp.where(mb, jnp.inf, d2)
        return 0

    lax.fori_loop(clo, chi, fill, 0)

    pv = jnp.full((R, 1), -jnp.inf, jnp.float32)
    pi = jnp.full((R, 1), -1, jnp.int32)
    for t in range(K):
        def scan(c, bc, pv=pv, pi=pi):
            bv, bi = bc
            Dc = D[c]
            ii = c * CW + lax.broadcasted_iota(jnp.int32, (R, CW), 1)
            valid = (Dc > pv) | ((Dc == pv) & (ii > pi))
            v = jnp.where(valid, Dc, jnp.inf)
            m = jnp.min(v, axis=1, keepdims=True)
            im = jnp.min(jnp.where(v == m, ii, IBIG), axis=1, keepdims=True)
            take = m < bv
            return jnp.where(take, m, bv), jnp.where(take, im, bi)

        bv, bi = lax.fori_loop(
            clo, chi, scan,
            (jnp.full((R, 1), jnp.inf, jnp.float32),
             jnp.full((R, 1), IBIG, jnp.int32)))
        bi = jnp.where(bv == jnp.inf, selfi, bi)
        out[:, t:t + 1] = bi
        pv, pi = bv, bi


def _knn_topk(pos, batch):
    N = pos.shape[0]
    nch = -(-N // CW)
    npad = nch * CW
    nrp = -(-N // R) * R
    b32 = batch.astype(jnp.int32)
    pos_r = jnp.pad(pos, ((0, nrp - N), (0, 0)))
    batR = jnp.pad(b32, (0, nrp - N), constant_values=126)[:, None]
    posT3 = jnp.pad(pos, ((0, npad - N), (0, 0))).T.reshape(3, nch, CW)
    posT3 = posT3.transpose(1, 0, 2)
    batT3 = jnp.pad(b32, (0, npad - N), constant_values=127).reshape(nch, 1, CW)
    grid = nrp // R
    seg = jnp.searchsorted(b32, jnp.arange(9), side="left").astype(jnp.int32)
    rb = jnp.arange(grid, dtype=jnp.int32)
    bpad = jnp.pad(b32, (0, nrp - N), constant_values=7)
    blo = jnp.clip(bpad[rb * R], 0, 7)
    bhi = jnp.clip(bpad[rb * R + R - 1], 0, 7)
    c_lo = seg[blo] // CW
    c_hi = -(-seg[bhi + 1] // CW)
    c_cnt = c_hi - c_lo
    return pl.pallas_call(
        functools.partial(_knn_body, nch),
        grid=(grid,),
        in_specs=[
            pl.BlockSpec(memory_space=pltpu.SMEM),
            pl.BlockSpec(memory_space=pltpu.SMEM),
            pl.BlockSpec((R, 3), lambda i: (i, 0)),
            pl.BlockSpec((nch, 3, CW), lambda i: (0, 0, 0)),
            pl.BlockSpec((nch, 1, CW), lambda i: (0, 0, 0)),
            pl.BlockSpec((R, 1), lambda i: (i, 0)),
        ],
        out_specs=pl.BlockSpec((R, K), lambda i: (i, 0)),
        out_shape=jax.ShapeDtypeStruct((nrp, K), jnp.int32),
        scratch_shapes=[pltpu.VMEM((nch, R, CW), jnp.float32)],
    )(c_lo, c_cnt, pos_r, posT3, batT3, batR)[:N]


def _sc_body(nqp, C, x_hbm, idxf_hbm, out_hbm, nbuf, qbuf, idxf, zbuf,
             var_sh, sem):
    c = lax.axis_index("c")
    s = lax.axis_index("s")
    wid = s * NC + c
    rows_per_tile = nqp // NS
    for r in range(LL):
        for ch in range(C // LL):
            zbuf[r, pl.ds(ch * LL, LL)] = jnp.zeros((LL,), jnp.float32)

    def zloop(z, _):
        pltpu.sync_copy(zbuf, var_sh.at[pl.ds(s * rows_per_tile + z * LL, LL)])
        return 0

    lax.fori_loop(0, rows_per_tile // LL, zloop, 0)
    plsc.subcore_barrier()

    qpw = nqp // (NC * NS)

    def giter(g, _):
        base = wid * qpw + g * QB
        pltpu.sync_copy(idxf_hbm.at[pl.ds(base * K, QB * K)], idxf)
        cp = pltpu.async_copy(x_hbm.at[idxf], nbuf, sem)
        pltpu.sync_copy(x_hbm.at[pl.ds(base, QB)], qbuf)
        cp.wait()

        def qloop(q, _):
            for ch in range(C // LL):
                qv = qbuf[q, pl.ds(ch * LL, LL)]
                for j in range(K):
                    d = nbuf[q * K + j, pl.ds(ch * LL, LL)] - qv
                    nbuf[q * K + j, pl.ds(ch * LL, LL)] = d * d
            return 0

        lax.fori_loop(0, QB, qloop, 0)
        pltpu.sync_copy(nbuf, var_sh.at[idxf], add=True)
        return 0

    lax.fori_loop(0, qpw // QB, giter, 0)
    plsc.subcore_barrier()
    pltpu.sync_copy(var_sh.at[pl.ds(s * rows_per_tile, rows_per_tile)],
                    out_hbm.at[c, pl.ds(s * rows_per_tile, rows_per_tile)])


def _sc_variance(x, idx):
    N, C = x.shape
    nqp = -(-N // (NC * NS * QB)) * (NC * NS * QB)
    x_pad = jnp.pad(x, ((0, nqp - N), (0, 0)))
    idx_pad = jnp.pad(idx, ((0, nqp - N), (0, 0)), constant_values=N)
    idx_flat = idx_pad.reshape(-1)
    mesh = plsc.VectorSubcoreMesh(core_axis_name="c", subcore_axis_name="s",
                                  num_cores=NC, num_subcores=NS)
    fn = pl.kernel(
        functools.partial(_sc_body, nqp, C),
        out_type=jax.ShapeDtypeStruct((NC, nqp, C), jnp.float32),
        mesh=mesh,
        scratch_types=[
            pltpu.VMEM((QB * K, C), jnp.float32),
            pltpu.VMEM((QB, C), jnp.float32),
            pltpu.VMEM((QB * K,), jnp.int32),
            pltpu.VMEM((LL, C), jnp.float32),
            pltpu.VMEM_SHARED((nqp, C), jnp.float32),
            pltpu.SemaphoreType.DMA,
        ],
    )
    return fn(x_pad, idx_flat)


def _mlp_body(var2, W1T, b1, w2, b2, out):
    v = (var2[0] + var2[1]) * jnp.float32(1.0 / K)
    h = jnp.dot(v, W1T[...], preferred_element_type=jnp.float32) + b1[...]
    h = jnp.maximum(h, 0.0)
    o = jnp.dot(h, w2[...], preferred_element_type=jnp.float32) + b2[0, 0]
    out[...] = jax.nn.sigmoid(o)


def _mlp(var2, W1, b1, W2, b2):
    _, nqp, C = var2.shape
    H = W1.shape[0]
    grid = nqp // MB
    return pl.pallas_call(
        _mlp_body,
        grid=(grid,),
        in_specs=[
            pl.BlockSpec((NC, MB, C), lambda i: (0, i, 0)),
            pl.BlockSpec((C, H), lambda i: (0, 0)),
            pl.BlockSpec((1, H), lambda i: (0, 0)),
            pl.BlockSpec((H, 1), lambda i: (0, 0)),
            pl.BlockSpec((1, 1), lambda i: (0, 0), memory_space=pltpu.SMEM),
        ],
        out_specs=pl.BlockSpec((MB, 1), lambda i: (i, 0)),
        out_shape=jax.ShapeDtypeStruct((nqp, 1), jnp.float32),
    )(var2, W1.T, b1[None, :], W2.T, b2[None, :])


def kernel(x, pos, batch, W1, b1, W2, b2):
    N = x.shape[0]
    idx = _knn_topk(pos, batch)
    var2 = _sc_variance(x, idx)
    return _mlp(var2, W1, b1, W2, b2)[:N]


# speedup vs baseline: 33.8672x; 1.7053x over previous
import functools

import jax
import jax.numpy as jnp
from jax import lax
from jax.experimental import pallas as pl
from jax.experimental.pallas import tpu as pltpu
from jax.experimental.pallas import tpu_sc as plsc

K = 16
R = 64
CW = 512
NC = 2
NS = 16
LL = 16
QB = 8
MB = 256
IBIG = 2**30


def _knn_body(nch, cl_ref, cn_ref, pos_r, posT3, batT3, batR, out, D):
    qx = pos_r[:, 0:1]
    qy = pos_r[:, 1:2]
    qz = pos_r[:, 2:3]
    qb = batR[:, 0:1]
    i = pl.program_id(0)
    selfi = i * R + lax.broadcasted_iota(jnp.int32, (R, 1), 0)
    clo = cl_ref[i]
    chi = clo + cn_ref[i]

    def fill(c, _):
        px = posT3[c, 0:1, :]
        py = posT3[c, 1:2, :]
        pz = posT3[c, 2:3, :]
        dx = qx - px
        dy = qy - py
        dz = qz - pz
        d2 = (dx * dx + dy * dy) + dz * dz
        mb = batT3[c] != qb
        D[c] = jnp.where(mb, jnp.inf, d2)
        return 0

    lax.fori_loop(clo, chi, fill, 0)

    pv = jnp.full((R, 1), -jnp.inf, jnp.float32)
    pi = jnp.full((R, 1), -1, jnp.int32)
    for t in range(K):
        def scan(c, bc, pv=pv, pi=pi):
            bv, bi = bc
            Dc = D[c]
            ii = c * CW + lax.broadcasted_iota(jnp.int32, (R, CW), 1)
            valid = (Dc > pv) | ((Dc == pv) & (ii > pi))
            v = jnp.where(valid, Dc, jnp.inf)
            m = jnp.min(v, axis=1, keepdims=True)
            im = jnp.min(jnp.where(v == m, ii, IBIG), axis=1, keepdims=True)
            take = m < bv
            return jnp.where(take, m, bv), jnp.where(take, im, bi)

        bv, bi = lax.fori_loop(
            clo, chi, scan,
            (jnp.full((R, 1), jnp.inf, jnp.float32),
             jnp.full((R, 1), IBIG, jnp.int32)))
        bi = jnp.where(bv == jnp.inf, selfi, bi)
        out[:, t:t + 1] = bi
        pv, pi = bv, bi


def _knn_topk(pos, batch):
    N = pos.shape[0]
    nch = -(-N // CW)
    npad = nch * CW
    nrp = -(-N // R) * R
    b32 = batch.astype(jnp.int32)
    pos_r = jnp.pad(pos, ((0, nrp - N), (0, 0)))
    batR = jnp.pad(b32, (0, nrp - N), constant_values=126)[:, None]
    posT3 = jnp.pad(pos, ((0, npad - N), (0, 0))).T.reshape(3, nch, CW)
    posT3 = posT3.transpose(1, 0, 2)
    batT3 = jnp.pad(b32, (0, npad - N), constant_values=127).reshape(nch, 1, CW)
    grid = nrp // R
    seg = jnp.searchsorted(b32, jnp.arange(9), side="left").astype(jnp.int32)
    rb = jnp.arange(grid, dtype=jnp.int32)
    bpad = jnp.pad(b32, (0, nrp - N), constant_values=7)
    blo = jnp.clip(bpad[rb * R], 0, 7)
    bhi = jnp.clip(bpad[rb * R + R - 1], 0, 7)
    c_lo = seg[blo] // CW
    c_hi = -(-seg[bhi + 1] // CW)
    c_cnt = c_hi - c_lo
    return pl.pallas_call(
        functools.partial(_knn_body, nch),
        grid=(grid,),
        in_specs=[
            pl.BlockSpec(memory_space=pltpu.SMEM),
            pl.BlockSpec(memory_space=pltpu.SMEM),
            pl.BlockSpec((R, 3), lambda i: (i, 0)),
            pl.BlockSpec((nch, 3, CW), lambda i: (0, 0, 0)),
            pl.BlockSpec((nch, 1, CW), lambda i: (0, 0, 0)),
            pl.BlockSpec((R, 1), lambda i: (i, 0)),
        ],
        out_specs=pl.BlockSpec((R, K), lambda i: (i, 0)),
        out_shape=jax.ShapeDtypeStruct((nrp, K), jnp.int32),
        scratch_shapes=[pltpu.VMEM((nch, R, CW), jnp.float32)],
    )(c_lo, c_cnt, pos_r, posT3, batT3, batR)[:N]


def _sc_body(nqp, C, x_hbm, idxf_hbm, out_hbm, nbuf, qbuf, idxf, zbuf,
             var_sh, sem):
    c = lax.axis_index("c")
    s = lax.axis_index("s")
    wid = s * NC + c
    rows_per_tile = nqp // NS
    for r in range(LL):
        for ch in range(C // LL):
            zbuf[r, pl.ds(ch * LL, LL)] = jnp.zeros((LL,), jnp.float32)

    def zloop(z, _):
        pltpu.sync_copy(zbuf, var_sh.at[pl.ds(s * rows_per_tile + z * LL, LL)])
        return 0

    lax.fori_loop(0, rows_per_tile // LL, zloop, 0)
    plsc.subcore_barrier()

    qpw = nqp // (NC * NS)

    def giter(g, _):
        base = wid * qpw + g * QB
        pltpu.sync_copy(idxf_hbm.at[pl.ds(base * K, QB * K)], idxf)
        cp = pltpu.async_copy(x_hbm.at[idxf], nbuf, sem)
        pltpu.sync_copy(x_hbm.at[pl.ds(base, QB)], qbuf)
        cp.wait()

        def qloop(q, _):
            for ch in range(C // LL):
                qv = qbuf[q, pl.ds(ch * LL, LL)]
                for j in range(K):
                    d = nbuf[q * K + j, pl.ds(ch * LL, LL)] - qv
                    nbuf[q * K + j, pl.ds(ch * LL, LL)] = d * d
            return 0

        lax.fori_loop(0, QB, qloop, 0)
        pltpu.sync_copy(nbuf, var_sh.at[idxf], add=True)
        return 0

    lax.fori_loop(0, qpw // QB, giter, 0)
    plsc.subcore_barrier()
    pltpu.sync_copy(var_sh.at[pl.ds(s * rows_per_tile, rows_per_tile)],
                    out_hbm.at[c, pl.ds(s * rows_per_tile, rows_per_tile)])


def _sc_variance(x, idx):
    N, C = x.shape
    nqp = -(-N // (NC * NS * QB)) * (NC * NS * QB)
    x_pad = jnp.pad(x, ((0, nqp - N), (0, 0)))
    idx_pad = jnp.pad(idx, ((0, nqp - N), (0, 0)), constant_values=N)
    idx_flat = idx_pad.reshape(-1)
    mesh = plsc.VectorSubcoreMesh(core_axis_name="c", subcore_axis_name="s",
                                  num_cores=NC, num_subcores=NS)
    fn = pl.kernel(
        functools.partial(_sc_body, nqp, C),
        out_type=jax.ShapeDtypeStruct((NC, nqp, C), jnp.float32),
        mesh=mesh,
        scratch_types=[
            pltpu.VMEM((QB * K, C), jnp.float32),
            pltpu.VMEM((QB, C), jnp.float32),
            pltpu.VMEM((QB * K,), jnp.int32),
            pltpu.VMEM((LL, C), jnp.float32),
            pltpu.VMEM_SHARED((nqp, C), jnp.float32),
            pltpu.SemaphoreType.DMA,
        ],
    )
    return fn(x_pad, idx_flat)


def _mlp_body(var2, W1T, b1, w2, b2, out):
    v = (var2[0] + var2[1]) * jnp.float32(1.0 / K)
    h = jnp.dot(v, W1T[...], preferred_element_type=jnp.float32) + b1[...]
    h = jnp.maximum(h, 0.0)
    o = jnp.dot(h, w2[...], preferred_element_type=jnp.float32) + b2[0, 0]
    out[...] = jax.nn.sigmoid(o)


def _mlp(var2, W1, b1, W2, b2):
    _, nqp, C = var2.shape
    H = W1.shape[0]
    grid = nqp // MB
    return pl.pallas_call(
        _mlp_body,
        grid=(grid,),
        in_specs=[
            pl.BlockSpec((NC, MB, C), lambda i: (0, i, 0)),
            pl.BlockSpec((C, H), lambda i: (0, 0)),
            pl.BlockSpec((1, H), lambda i: (0, 0)),
            pl.BlockSpec((H, 1), lambda i: (0, 0)),
            pl.BlockSpec((1, 1), lambda i: (0, 0), memory_space=pltpu.SMEM),
        ],
        out_specs=pl.BlockSpec((MB, 1), lambda i: (i, 0)),
        out_shape=jax.ShapeDtypeStruct((nqp, 1), jnp.float32),
    )(var2, W1.T, b1[None, :], W2.T, b2[None, :])


def kernel(x, pos, batch, W1, b1, W2, b2):
    N = x.shape[0]
    idx = _knn_topk(pos, batch)
    var2 = _sc_variance(x, idx)
    return _mlp(var2, W1, b1, W2, b2)[:N]


# speedup vs baseline: 66.9415x; 1.9766x over previous
import functools

import jax
import jax.numpy as jnp
from jax import lax
from jax.experimental import pallas as pl
from jax.experimental.pallas import tpu as pltpu
from jax.experimental.pallas import tpu_sc as plsc

K = 16
R = 128
CW = 512
NC = 2
NS = 16
LL = 16
QB = 8
MB = 256
IBIG = 2**30


def _knn_body(nch, cl_ref, cn_ref, pos_r, posT3, batT3, batR, out, D):
    qx = pos_r[:, 0:1]
    qy = pos_r[:, 1:2]
    qz = pos_r[:, 2:3]
    qb = batR[:, 0:1]
    i = pl.program_id(0)
    selfi = i * R + lax.broadcasted_iota(jnp.int32, (R, 1), 0)
    clo = cl_ref[i]
    chi = clo + cn_ref[i]

    def fill(c, _):
        px = posT3[c, 0:1, :]
        py = posT3[c, 1:2, :]
        pz = posT3[c, 2:3, :]
        dx = qx - px
        dy = qy - py
        dz = qz - pz
        d2 = (dx * dx + dy * dy) + dz * dz
        mb = batT3[c] != qb
        D[c] = jnp.where(mb, jnp.inf, d2)
        return 0

    lax.fori_loop(clo, chi, fill, 0)

    pv = jnp.full((R, 1), -jnp.inf, jnp.float32)
    pi = jnp.full((R, 1), -1, jnp.int32)
    for t in range(K):
        def scan(c, bc, pv=pv, pi=pi):
            bv, bi = bc
            Dc = D[c]
            ii = c * CW + lax.broadcasted_iota(jnp.int32, (R, CW), 1)
            valid = (Dc > pv) | ((Dc == pv) & (ii > pi))
            v = jnp.where(valid, Dc, jnp.inf)
            m = jnp.min(v, axis=1, keepdims=True)
            im = jnp.min(jnp.where(v == m, ii, IBIG), axis=1, keepdims=True)
            take = m < bv
            return jnp.where(take, m, bv), jnp.where(take, im, bi)

        bv, bi = lax.fori_loop(
            clo, chi, scan,
            (jnp.full((R, 1), jnp.inf, jnp.float32),
             jnp.full((R, 1), IBIG, jnp.int32)))
        bi = jnp.where(bv == jnp.inf, selfi, bi)
        out[:, t:t + 1] = bi
        pv, pi = bv, bi


def _knn_topk(pos, batch):
    N = pos.shape[0]
    nch = -(-N // CW)
    npad = nch * CW
    nrp = -(-N // R) * R
    b32 = batch.astype(jnp.int32)
    pos_r = jnp.pad(pos, ((0, nrp - N), (0, 0)))
    batR = jnp.pad(b32, (0, nrp - N), constant_values=126)[:, None]
    posT3 = jnp.pad(pos, ((0, npad - N), (0, 0))).T.reshape(3, nch, CW)
    posT3 = posT3.transpose(1, 0, 2)
    batT3 = jnp.pad(b32, (0, npad - N), constant_values=127).reshape(nch, 1, CW)
    grid = nrp // R
    seg = jnp.searchsorted(b32, jnp.arange(9), side="left").astype(jnp.int32)
    rb = jnp.arange(grid, dtype=jnp.int32)
    bpad = jnp.pad(b32, (0, nrp - N), constant_values=7)
    blo = jnp.clip(bpad[rb * R], 0, 7)
    bhi = jnp.clip(bpad[rb * R + R - 1], 0, 7)
    c_lo = seg[blo] // CW
    c_hi = -(-seg[bhi + 1] // CW)
    c_cnt = c_hi - c_lo
    return pl.pallas_call(
        functools.partial(_knn_body, nch),
        grid=(grid,),
        in_specs=[
            pl.BlockSpec(memory_space=pltpu.SMEM),
            pl.BlockSpec(memory_space=pltpu.SMEM),
            pl.BlockSpec((R, 3), lambda i: (i, 0)),
            pl.BlockSpec((nch, 3, CW), lambda i: (0, 0, 0)),
            pl.BlockSpec((nch, 1, CW), lambda i: (0, 0, 0)),
            pl.BlockSpec((R, 1), lambda i: (i, 0)),
        ],
        out_specs=pl.BlockSpec((R, K), lambda i: (i, 0)),
        out_shape=jax.ShapeDtypeStruct((nrp, K), jnp.int32),
        scratch_shapes=[pltpu.VMEM((nch, R, CW), jnp.float32)],
    )(c_lo, c_cnt, pos_r, posT3, batT3, batR)[:N]


def _sc_body(nqp, C, x_hbm, idxf_hbm, out_hbm, nbuf, qbuf, idxf, zbuf,
             var_sh, sem):
    c = lax.axis_index("c")
    s = lax.axis_index("s")
    wid = s * NC + c
    rows_per_tile = nqp // NS
    for r in range(LL):
        for ch in range(C // LL):
            zbuf[r, pl.ds(ch * LL, LL)] = jnp.zeros((LL,), jnp.float32)

    def zloop(z, _):
        pltpu.sync_copy(zbuf, var_sh.at[pl.ds(s * rows_per_tile + z * LL, LL)])
        return 0

    lax.fori_loop(0, rows_per_tile // LL, zloop, 0)
    plsc.subcore_barrier()

    qpw = nqp // (NC * NS)

    def giter(g, _):
        base = wid * qpw + g * QB
        pltpu.sync_copy(idxf_hbm.at[pl.ds(base * K, QB * K)], idxf)
        cp = pltpu.async_copy(x_hbm.at[idxf], nbuf, sem)
        pltpu.sync_copy(x_hbm.at[pl.ds(base, QB)], qbuf)
        cp.wait()

        def qloop(q, _):
            for ch in range(C // LL):
                qv = qbuf[q, pl.ds(ch * LL, LL)]
                for j in range(K):
                    d = nbuf[q * K + j, pl.ds(ch * LL, LL)] - qv
                    nbuf[q * K + j, pl.ds(ch * LL, LL)] = d * d
            return 0

        lax.fori_loop(0, QB, qloop, 0)
        pltpu.sync_copy(nbuf, var_sh.at[idxf], add=True)
        return 0

    lax.fori_loop(0, qpw // QB, giter, 0)
    plsc.subcore_barrier()
    pltpu.sync_copy(var_sh.at[pl.ds(s * rows_per_tile, rows_per_tile)],
                    out_hbm.at[c, pl.ds(s * rows_per_tile, rows_per_tile)])


def _sc_variance(x, idx):
    N, C = x.shape
    nqp = -(-N // (NC * NS * QB)) * (NC * NS * QB)
    x_pad = jnp.pad(x, ((0, nqp - N), (0, 0)))
    idx_pad = jnp.pad(idx, ((0, nqp - N), (0, 0)), constant_values=N)
    idx_flat = idx_pad.reshape(-1)
    mesh = plsc.VectorSubcoreMesh(core_axis_name="c", subcore_axis_name="s",
                                  num_cores=NC, num_subcores=NS)
    fn = pl.kernel(
        functools.partial(_sc_body, nqp, C),
        out_type=jax.ShapeDtypeStruct((NC, nqp, C), jnp.float32),
        mesh=mesh,
        scratch_types=[
            pltpu.VMEM((QB * K, C), jnp.float32),
            pltpu.VMEM((QB, C), jnp.float32),
            pltpu.VMEM((QB * K,), jnp.int32),
            pltpu.VMEM((LL, C), jnp.float32),
            pltpu.VMEM_SHARED((nqp, C), jnp.float32),
            pltpu.SemaphoreType.DMA,
        ],
    )
    return fn(x_pad, idx_flat)


def _mlp_body(var2, W1T, b1, w2, b2, out):
    v = (var2[0] + var2[1]) * jnp.float32(1.0 / K)
    h = jnp.dot(v, W1T[...], preferred_element_type=jnp.float32) + b1[...]
    h = jnp.maximum(h, 0.0)
    o = jnp.dot(h, w2[...], preferred_element_type=jnp.float32) + b2[0, 0]
    out[...] = jax.nn.sigmoid(o)


def _mlp(var2, W1, b1, W2, b2):
    _, nqp, C = var2.shape
    H = W1.shape[0]
    grid = nqp // MB
    return pl.pallas_call(
        _mlp_body,
        grid=(grid,),
        in_specs=[
            pl.BlockSpec((NC, MB, C), lambda i: (0, i, 0)),
            pl.BlockSpec((C, H), lambda i: (0, 0)),
            pl.BlockSpec((1, H), lambda i: (0, 0)),
            pl.BlockSpec((H, 1), lambda i: (0, 0)),
            pl.BlockSpec((1, 1), lambda i: (0, 0), memory_space=pltpu.SMEM),
        ],
        out_specs=pl.BlockSpec((MB, 1), lambda i: (i, 0)),
        out_shape=jax.ShapeDtypeStruct((nqp, 1), jnp.float32),
    )(var2, W1.T, b1[None, :], W2.T, b2[None, :])


def kernel(x, pos, batch, W1, b1, W2, b2):
    N = x.shape[0]
    idx = _knn_topk(pos, batch)
    var2 = _sc_variance(x, idx)
    return _mlp(var2, W1, b1, W2, b2)[:N]


# speedup vs baseline: 67.0254x; 1.0013x over previous
import functools

import jax
import jax.numpy as jnp
from jax import lax
from jax.experimental import pallas as pl
from jax.experimental.pallas import tpu as pltpu
from jax.experimental.pallas import tpu_sc as plsc

K = 16
R = 128
CW = 512
NC = 2
NS = 16
LL = 16
QB = 8
MB = 256
IBIG = 2**30


def _knn_body(nch, cl_ref, cn_ref, pos_r, posT3, batT3, batR, out, D):
    qx = pos_r[:, 0:1]
    qy = pos_r[:, 1:2]
    qz = pos_r[:, 2:3]
    qb = batR[:, 0:1]
    i = pl.program_id(0)
    selfi = i * R + lax.broadcasted_iota(jnp.int32, (R, 1), 0)
    clo = cl_ref[i]
    chi = clo + cn_ref[i]
    iota2 = lax.broadcasted_iota(jnp.int32, (R, CW), 1)

    def fill(c, M):
        px = posT3[c, 0:1, :]
        py = posT3[c, 1:2, :]
        pz = posT3[c, 2:3, :]
        dx = qx - px
        dy = qy - py
        dz = qz - pz
        d2 = (dx * dx + dy * dy) + dz * dz
        mb = batT3[c] != qb
        v = jnp.where(mb, jnp.inf, d2)
        D[c] = v
        return jnp.minimum(M, v)

    M = lax.fori_loop(clo, chi, fill,
                      jnp.full((R, CW), jnp.inf, jnp.float32))

    pv = None
    pi = None
    for t in range(K):
        if t > 0:
            def scan1(c, M, pv=pv, pi=pi):
                Dc = D[c]
                ii = c * CW + iota2
                kill = (Dc == pv) & (ii == pi)
                v = jnp.where(kill, jnp.inf, Dc)
                D[c] = v
                return jnp.minimum(M, v)

            M = lax.fori_loop(clo, chi, scan1,
                              jnp.full((R, CW), jnp.inf, jnp.float32))
        m = jnp.min(M, axis=1, keepdims=True)

        def scan2(c, A, m=m):
            Dc = D[c]
            ii = c * CW + iota2
            return jnp.minimum(A, jnp.where(Dc == m, ii, IBIG))

        A = lax.fori_loop(clo, chi, scan2,
                          jnp.full((R, CW), IBIG, jnp.int32))
        im = jnp.min(A, axis=1, keepdims=True)
        im = jnp.where(m == jnp.inf, selfi, im)
        out[:, t:t + 1] = im
        pv, pi = m, im


def _knn_topk(pos, batch):
    N = pos.shape[0]
    nch = -(-N // CW)
    npad = nch * CW
    nrp = -(-N // R) * R
    b32 = batch.astype(jnp.int32)
    pos_r = jnp.pad(pos, ((0, nrp - N), (0, 0)))
    batR = jnp.pad(b32, (0, nrp - N), constant_values=126)[:, None]
    posT3 = jnp.pad(pos, ((0, npad - N), (0, 0))).T.reshape(3, nch, CW)
    posT3 = posT3.transpose(1, 0, 2)
    batT3 = jnp.pad(b32, (0, npad - N), constant_values=127).reshape(nch, 1, CW)
    grid = nrp // R
    seg = jnp.searchsorted(b32, jnp.arange(9), side="left").astype(jnp.int32)
    rb = jnp.arange(grid, dtype=jnp.int32)
    bpad = jnp.pad(b32, (0, nrp - N), constant_values=7)
    blo = jnp.clip(bpad[rb * R], 0, 7)
    bhi = jnp.clip(bpad[rb * R + R - 1], 0, 7)
    c_lo = seg[blo] // CW
    c_hi = -(-seg[bhi + 1] // CW)
    c_cnt = c_hi - c_lo
    return pl.pallas_call(
        functools.partial(_knn_body, nch),
        grid=(grid,),
        in_specs=[
            pl.BlockSpec(memory_space=pltpu.SMEM),
            pl.BlockSpec(memory_space=pltpu.SMEM),
            pl.BlockSpec((R, 3), lambda i: (i, 0)),
            pl.BlockSpec((nch, 3, CW), lambda i: (0, 0, 0)),
            pl.BlockSpec((nch, 1, CW), lambda i: (0, 0, 0)),
            pl.BlockSpec((R, 1), lambda i: (i, 0)),
        ],
        out_specs=pl.BlockSpec((R, K), lambda i: (i, 0)),
        out_shape=jax.ShapeDtypeStruct((nrp, K), jnp.int32),
        scratch_shapes=[pltpu.VMEM((nch, R, CW), jnp.float32)],
    )(c_lo, c_cnt, pos_r, posT3, batT3, batR)[:N]


def _sc_body(nqp, C, x_hbm, idxf_hbm, out_hbm, nbuf, qbuf, idxf, zbuf,
             var_sh, sem):
    c = lax.axis_index("c")
    s = lax.axis_index("s")
    wid = s * NC + c
    rows_per_tile = nqp // NS
    for r in range(LL):
        for ch in range(C // LL):
            zbuf[r, pl.ds(ch * LL, LL)] = jnp.zeros((LL,), jnp.float32)

    def zloop(z, _):
        pltpu.sync_copy(zbuf, var_sh.at[pl.ds(s * rows_per_tile + z * LL, LL)])
        return 0

    lax.fori_loop(0, rows_per_tile // LL, zloop, 0)
    plsc.subcore_barrier()

    qpw = nqp // (NC * NS)

    def giter(g, _):
        base = wid * qpw + g * QB
        pltpu.sync_copy(idxf_hbm.at[pl.ds(base * K, QB * K)], idxf)
        cp = pltpu.async_copy(x_hbm.at[idxf], nbuf, sem)
        pltpu.sync_copy(x_hbm.at[pl.ds(base, QB)], qbuf)
        cp.wait()

        def qloop(q, _):
            for ch in range(C // LL):
                qv = qbuf[q, pl.ds(ch * LL, LL)]
                for j in range(K):
                    d = nbuf[q * K + j, pl.ds(ch * LL, LL)] - qv
                    nbuf[q * K + j, pl.ds(ch * LL, LL)] = d * d
            return 0

        lax.fori_loop(0, QB, qloop, 0)
        pltpu.sync_copy(nbuf, var_sh.at[idxf], add=True)
        return 0

    lax.fori_loop(0, qpw // QB, giter, 0)
    plsc.subcore_barrier()
    pltpu.sync_copy(var_sh.at[pl.ds(s * rows_per_tile, rows_per_tile)],
                    out_hbm.at[c, pl.ds(s * rows_per_tile, rows_per_tile)])


def _sc_variance(x, idx):
    N, C = x.shape
    nqp = -(-N // (NC * NS * QB)) * (NC * NS * QB)
    x_pad = jnp.pad(x, ((0, nqp - N), (0, 0)))
    idx_pad = jnp.pad(idx, ((0, nqp - N), (0, 0)), constant_values=N)
    idx_flat = idx_pad.reshape(-1)
    mesh = plsc.VectorSubcoreMesh(core_axis_name="c", subcore_axis_name="s",
                                  num_cores=NC, num_subcores=NS)
    fn = pl.kernel(
        functools.partial(_sc_body, nqp, C),
        out_type=jax.ShapeDtypeStruct((NC, nqp, C), jnp.float32),
        mesh=mesh,
        scratch_types=[
            pltpu.VMEM((QB * K, C), jnp.float32),
            pltpu.VMEM((QB, C), jnp.float32),
            pltpu.VMEM((QB * K,), jnp.int32),
            pltpu.VMEM((LL, C), jnp.float32),
            pltpu.VMEM_SHARED((nqp, C), jnp.float32),
            pltpu.SemaphoreType.DMA,
        ],
    )
    return fn(x_pad, idx_flat)


def _mlp_body(var2, W1T, b1, w2, b2, out):
    v = (var2[0] + var2[1]) * jnp.float32(1.0 / K)
    h = jnp.dot(v, W1T[...], preferred_element_type=jnp.float32) + b1[...]
    h = jnp.maximum(h, 0.0)
    o = jnp.dot(h, w2[...], preferred_element_type=jnp.float32) + b2[0, 0]
    out[...] = jax.nn.sigmoid(o)


def _mlp(var2, W1, b1, W2, b2):
    _, nqp, C = var2.shape
    H = W1.shape[0]
    grid = nqp // MB
    return pl.pallas_call(
        _mlp_body,
        grid=(grid,),
        in_specs=[
            pl.BlockSpec((NC, MB, C), lambda i: (0, i, 0)),
            pl.BlockSpec((C, H), lambda i: (0, 0)),
            pl.BlockSpec((1, H), lambda i: (0, 0)),
            pl.BlockSpec((H, 1), lambda i: (0, 0)),
            pl.BlockSpec((1, 1), lambda i: (0, 0), memory_space=pltpu.SMEM),
        ],
        out_specs=pl.BlockSpec((MB, 1), lambda i: (i, 0)),
        out_shape=jax.ShapeDtypeStruct((nqp, 1), jnp.float32),
    )(var2, W1.T, b1[None, :], W2.T, b2[None, :])


def kernel(x, pos, batch, W1, b1, W2, b2):
    N = x.shape[0]
    idx = _knn_topk(pos, batch)
    var2 = _sc_variance(x, idx)
    return _mlp(var2, W1, b1, W2, b2)[:N]
